# baseline (speedup 1.0000x reference)
import functools

import jax
import jax.numpy as jnp
from jax import lax
from jax.experimental import pallas as pl
from jax.experimental.pallas import tpu as pltpu
from jax.experimental.pallas import tpu_sc as plsc

B = 512
NUM_EDGES = 8192
NUM_NODES = 4096
PRE_LEN = 4
NUM_DIR = 8
MAX_ADJ = 16

NC = 2
NS = 16
L = 16
NW = NC * NS
RPW = B // NW

_BB = 64


def _tc_stats_body(x_ref, d_ref, mx_ref, lse_ref, am_ref, vd_ref):
    ms, lses, ams, vds = [], [], [], []
    for d in range(PRE_LEN):
        x = x_ref[:, d * NUM_EDGES:(d + 1) * NUM_EDGES]
        m = jnp.max(x, axis=1)
        lses.append(jnp.log(jnp.sum(jnp.exp(x - m[:, None]), axis=1))[:, None])
        ms.append(m[:, None])
        xd = d_ref[:, d * NUM_DIR:(d + 1) * NUM_DIR]
        md = jnp.max(xd, axis=1)
        vds.append(-jnp.log(jnp.sum(jnp.exp(xd - md[:, None]), axis=1))[:, None])
        lane = lax.broadcasted_iota(jnp.int32, xd.shape, 1)
        am = jnp.min(jnp.where(xd == md[:, None], lane, jnp.int32(NUM_DIR)),
                     axis=1)
        ams.append(am[:, None])
    mx_ref[...] = jnp.concatenate(ms, axis=1)
    lse_ref[...] = jnp.concatenate(lses, axis=1)
    am_ref[...] = jnp.concatenate(ams, axis=1)
    vd_ref[...] = jnp.concatenate(vds, axis=1)


def _tc_stats(pred, pred_d):
    out4 = jax.ShapeDtypeStruct((B, PRE_LEN), jnp.float32)
    out4i = jax.ShapeDtypeStruct((B, PRE_LEN), jnp.int32)
    return pl.pallas_call(
        _tc_stats_body,
        grid=(B // _BB,),
        in_specs=[
            pl.BlockSpec((_BB, PRE_LEN * NUM_EDGES), lambda i: (i, 0)),
            pl.BlockSpec((_BB, PRE_LEN * NUM_DIR), lambda i: (i, 0)),
        ],
        out_specs=[pl.BlockSpec((_BB, PRE_LEN), lambda i: (i, 0))] * 4,
        out_shape=[out4, out4, out4i, out4],
    )(pred, pred_d)


def _sc_chain_body(predf, gt, obs, ge, adj, offset,
                   outp_hbm, outv_hbm,
                   idx_v, en_v, naj_v, pidx_v, pval_v, najm_v,
                   ge_v, gt_v, off_v, outp_v, outv_v, sem, sem2):
    c = lax.axis_index("c")
    s = lax.axis_index("s")
    wid = s * NC + c
    base = wid * RPW
    lanes = lax.iota(jnp.int32, L)
    zeros = jnp.zeros((L,), jnp.int32)
    bvec = base + lanes

    cp_ge = pltpu.async_copy(ge, ge_v, sem2)
    pltpu.sync_copy(offset, off_v)
    pltpu.sync_copy(gt.at[pl.ds(base, RPW)], gt_v)
    pltpu.sync_copy(obs.at[pl.ds(base, RPW)], idx_v)
    last_pred = idx_v[...]
    padv = jnp.int32(NUM_EDGES) - plsc.load_gather(off_v, [zeros])
    cp_ge.wait()
    gt0 = plsc.load_gather(gt_v, [lanes, zeros])
    en_v[...] = plsc.load_gather(ge_v, [gt0, zeros]) - 1

    for dim in range(PRE_LEN):
        pltpu.async_copy(adj.at[en_v], naj_v, sem).wait()
        for j in range(MAX_ADJ):
            col = plsc.load_gather(
                naj_v, [lanes, jnp.full((L,), j, jnp.int32)])
            pad = (col == jnp.int32(NUM_EDGES)) | (col == last_pred)
            najm = jnp.where(pad, jnp.int32(NUM_EDGES), col)
            a, off = j // 8, (j % 8) * L
            najm_v[a, pl.ds(off, L)] = najm
            cc = jnp.int32(dim * NUM_EDGES) + jnp.where(pad, jnp.int32(0), col)
            flat = (((bvec >> 3) << 18) | ((cc >> 7) << 10)
                    | ((bvec & 7) << 7) | (cc & 127))
            pidx_v[a, pl.ds(off, L)] = flat
        c0 = pltpu.async_copy(predf.at[pidx_v.at[0]], pval_v.at[0], sem)
        c1 = pltpu.async_copy(predf.at[pidx_v.at[1]], pval_v.at[1], sem2)
        c0.wait()
        c1.wait()
        best_v = None
        best_n = None
        for j in range(MAX_ADJ):
            a, off = j // 8, (j % 8) * L
            najm = najm_v[a, pl.ds(off, L)]
            v = pval_v[a, pl.ds(off, L)]
            v = jnp.where(najm == jnp.int32(NUM_EDGES), -jnp.inf, v)
            if j == 0:
                best_v, best_n = v, najm
            else:
                upd = v > best_v
                best_v = jnp.where(upd, v, best_v)
                best_n = jnp.where(upd, najm, best_n)
        cur = jnp.where(best_n == jnp.int32(NUM_EDGES), padv, best_n)
        outp_v[dim, :] = cur
        outv_v[dim, :] = best_v
        last_pred = cur
        if dim != PRE_LEN - 1:
            en_v[...] = plsc.load_gather(
                ge_v, [cur, jnp.full((L,), 1, jnp.int32)]) - 1

    pltpu.sync_copy(outp_v, outp_hbm.at[:, pl.ds(base, RPW)])
    pltpu.sync_copy(outv_v, outv_hbm.at[:, pl.ds(base, RPW)])


def _sc_chain(predf, gt, obs, ge, adj, offset):
    mesh = plsc.VectorSubcoreMesh(core_axis_name="c", subcore_axis_name="s",
                                  num_cores=NC, num_subcores=NS)
    fn = pl.kernel(
        _sc_chain_body,
        compiler_params=pltpu.CompilerParams(needs_layout_passes=False,
                                             use_tc_tiling_on_sc=False),
        out_type=[
            jax.ShapeDtypeStruct((PRE_LEN, B), jnp.int32),
            jax.ShapeDtypeStruct((PRE_LEN, B), jnp.float32),
        ],
        mesh=mesh,
        scratch_types=[
            pltpu.VMEM((RPW,), jnp.int32),
            pltpu.VMEM((RPW,), jnp.int32),
            pltpu.VMEM((RPW, MAX_ADJ), jnp.int32),
            pltpu.VMEM((2, 128), jnp.int32),
            pltpu.VMEM((2, 128), jnp.float32),
            pltpu.VMEM((2, 128), jnp.int32),
            pltpu.VMEM((NUM_EDGES, 3), jnp.int32),
            pltpu.VMEM((RPW, PRE_LEN), jnp.int32),
            pltpu.VMEM((1,), jnp.int32),
            pltpu.VMEM((PRE_LEN, RPW), jnp.int32),
            pltpu.VMEM((PRE_LEN, RPW), jnp.float32),
            pltpu.SemaphoreType.DMA,
            pltpu.SemaphoreType.DMA,
        ],
    )
    return fn(predf, gt, obs, ge, adj, offset)


def _combine_body(predT_ref, rawT_ref, mx_ref, lse_ref, pred_ref, val_ref):
    pred_ref[...] = predT_ref[...].T
    val_ref[...] = (rawT_ref[...].T - mx_ref[...]) - lse_ref[...]


def _combine(predT, rawT, mx, lse):
    return pl.pallas_call(
        _combine_body,
        out_shape=[
            jax.ShapeDtypeStruct((B, PRE_LEN), jnp.int32),
            jax.ShapeDtypeStruct((B, PRE_LEN), jnp.float32),
        ],
    )(predT, rawT, mx, lse)


def kernel(pred, pred_d, gt, direction_gt, obs, graph_edges, node_adj_edges,
           offset):
    mx, lse, am, vd = _tc_stats(pred, pred_d)

    width = PRE_LEN * NUM_EDGES
    predf = (pred.reshape(B // 8, 8, width // 128, 128)
             .transpose(0, 2, 1, 3).reshape(B * width))
    predT, rawT = _sc_chain(predf, gt, obs, graph_edges, node_adj_edges,
                            offset)
    prediction, values = _combine(predT, rawT, mx, lse)
    return prediction, am, values, vd


# speedup vs baseline: 1.3300x; 1.3300x over previous
import functools

import jax
import jax.numpy as jnp
from jax import lax
from jax.experimental import pallas as pl
from jax.experimental.pallas import tpu as pltpu
from jax.experimental.pallas import tpu_sc as plsc

B = 512
NUM_EDGES = 8192
NUM_NODES = 4096
PRE_LEN = 4
NUM_DIR = 8
MAX_ADJ = 16

NC = 2
NS = 16
L = 16
NW = NC * NS
RPW = B // NW

_BB = 64


def _tc_stats_body(x_ref, d_ref, mx_ref, lse_ref, am_ref, vd_ref):
    ms, lses, ams, vds = [], [], [], []
    for d in range(PRE_LEN):
        x = x_ref[:, d * NUM_EDGES:(d + 1) * NUM_EDGES]
        m = jnp.max(x, axis=1)
        lses.append(jnp.log(jnp.sum(jnp.exp(x - m[:, None]), axis=1))[:, None])
        ms.append(m[:, None])
        xd = d_ref[:, d * NUM_DIR:(d + 1) * NUM_DIR]
        md = jnp.max(xd, axis=1)
        vds.append(-jnp.log(jnp.sum(jnp.exp(xd - md[:, None]), axis=1))[:, None])
        lane = lax.broadcasted_iota(jnp.int32, xd.shape, 1)
        am = jnp.min(jnp.where(xd == md[:, None], lane, jnp.int32(NUM_DIR)),
                     axis=1)
        ams.append(am[:, None])
    mx_ref[...] = jnp.concatenate(ms, axis=1)
    lse_ref[...] = jnp.concatenate(lses, axis=1)
    am_ref[...] = jnp.concatenate(ams, axis=1)
    vd_ref[...] = jnp.concatenate(vds, axis=1)


def _tc_stats(pred, pred_d):
    out4 = jax.ShapeDtypeStruct((B, PRE_LEN), jnp.float32)
    out4i = jax.ShapeDtypeStruct((B, PRE_LEN), jnp.int32)
    return pl.pallas_call(
        _tc_stats_body,
        grid=(B // _BB,),
        in_specs=[
            pl.BlockSpec((_BB, PRE_LEN * NUM_EDGES), lambda i: (i, 0)),
            pl.BlockSpec((_BB, PRE_LEN * NUM_DIR), lambda i: (i, 0)),
        ],
        out_specs=[pl.BlockSpec((_BB, PRE_LEN), lambda i: (i, 0))] * 4,
        out_shape=[out4, out4, out4i, out4],
    )(pred, pred_d)


def _sc_chain_body(predf, gt0, obs, ge0, ge1, adj, padval,
                   outp_hbm, outv_hbm,
                   idx_v, en_v, naj_v, pidx_v, pval_v, najm_v,
                   padv_v, outp_v, outv_v, sem, sem2):
    c = lax.axis_index("c")
    s = lax.axis_index("s")
    wid = s * NC + c
    base = wid * RPW

    pltpu.sync_copy(padval, padv_v)
    pltpu.sync_copy(gt0.at[pl.ds(base, RPW)], idx_v)
    pltpu.async_copy(ge0.at[idx_v], en_v, sem).wait()
    en_v[...] = en_v[...] - 1
    pltpu.sync_copy(obs.at[pl.ds(base, RPW)], idx_v)
    last_pred = idx_v[...]
    padv = padv_v[...]
    lanes = lax.iota(jnp.int32, L)
    bvec = base + lanes

    for dim in range(PRE_LEN):
        pltpu.async_copy(adj.at[en_v], naj_v, sem).wait()
        for j in range(MAX_ADJ):
            col = plsc.load_gather(
                naj_v, [lanes, jnp.full((L,), j, jnp.int32)])
            pad = (col == jnp.int32(NUM_EDGES)) | (col == last_pred)
            najm = jnp.where(pad, jnp.int32(NUM_EDGES), col)
            a, off = j // 8, (j % 8) * L
            najm_v[a, pl.ds(off, L)] = najm
            cc = jnp.int32(dim * NUM_EDGES) + jnp.where(pad, jnp.int32(0), col)
            flat = (((bvec >> 3) << 18) | ((cc >> 7) << 10)
                    | ((bvec & 7) << 7) | (cc & 127))
            pidx_v[a, pl.ds(off, L)] = flat
        c0 = pltpu.async_copy(predf.at[pidx_v.at[0]], pval_v.at[0], sem)
        c1 = pltpu.async_copy(predf.at[pidx_v.at[1]], pval_v.at[1], sem2)
        c0.wait()
        c1.wait()
        best_v = None
        best_n = None
        for j in range(MAX_ADJ):
            a, off = j // 8, (j % 8) * L
            najm = najm_v[a, pl.ds(off, L)]
            v = pval_v[a, pl.ds(off, L)]
            v = jnp.where(najm == jnp.int32(NUM_EDGES), -jnp.inf, v)
            if j == 0:
                best_v, best_n = v, najm
            else:
                upd = v > best_v
                best_v = jnp.where(upd, v, best_v)
                best_n = jnp.where(upd, najm, best_n)
        cur = jnp.where(best_n == jnp.int32(NUM_EDGES), padv, best_n)
        outp_v[dim, :] = cur
        outv_v[dim, :] = best_v
        last_pred = cur
        if dim != PRE_LEN - 1:
            idx_v[...] = cur
            pltpu.async_copy(ge1.at[idx_v], en_v, sem).wait()
            en_v[...] = en_v[...] - 1

    pltpu.sync_copy(outp_v, outp_hbm.at[:, pl.ds(base, RPW)])
    pltpu.sync_copy(outv_v, outv_hbm.at[:, pl.ds(base, RPW)])


def _sc_chain(predf, gt0, obs, ge0, ge1, adj, padval):
    mesh = plsc.VectorSubcoreMesh(core_axis_name="c", subcore_axis_name="s",
                                  num_cores=NC, num_subcores=NS)
    fn = pl.kernel(
        _sc_chain_body,
        compiler_params=pltpu.CompilerParams(needs_layout_passes=False,
                                             use_tc_tiling_on_sc=False),
        out_type=[
            jax.ShapeDtypeStruct((PRE_LEN, B), jnp.int32),
            jax.ShapeDtypeStruct((PRE_LEN, B), jnp.float32),
        ],
        mesh=mesh,
        scratch_types=[
            pltpu.VMEM((RPW,), jnp.int32),
            pltpu.VMEM((RPW,), jnp.int32),
            pltpu.VMEM((RPW, MAX_ADJ), jnp.int32),
            pltpu.VMEM((2, 128), jnp.int32),
            pltpu.VMEM((2, 128), jnp.float32),
            pltpu.VMEM((2, 128), jnp.int32),
            pltpu.VMEM((L,), jnp.int32),
            pltpu.VMEM((PRE_LEN, RPW), jnp.int32),
            pltpu.VMEM((PRE_LEN, RPW), jnp.float32),
            pltpu.SemaphoreType.DMA,
            pltpu.SemaphoreType.DMA,
        ],
    )
    return fn(predf, gt0, obs, ge0, ge1, adj, padval)


def kernel(pred, pred_d, gt, direction_gt, obs, graph_edges, node_adj_edges,
           offset):
    mx, lse, am, vd = _tc_stats(pred, pred_d)

    width = PRE_LEN * NUM_EDGES
    predf = (pred.reshape(B // 8, 8, width // 128, 128)
             .transpose(0, 2, 1, 3).reshape(B * width))
    gt0 = gt[:, 0].astype(jnp.int32)
    ge0 = graph_edges[:, 0].astype(jnp.int32)
    ge1 = graph_edges[:, 1].astype(jnp.int32)
    adj = node_adj_edges.astype(jnp.int32)
    padval = jnp.full((L,), NUM_EDGES, jnp.int32) - offset.astype(jnp.int32)[0]
    predT, rawT = _sc_chain(predf, gt0, obs.astype(jnp.int32), ge0, ge1, adj,
                            padval)

    prediction = predT.T
    values = (rawT.T - mx) - lse
    prediction_d = am
    values_d = vd
    return prediction, prediction_d, values, values_d


# speedup vs baseline: 1.3340x; 1.0030x over previous
import functools

import jax
import jax.numpy as jnp
from jax import lax
from jax.experimental import pallas as pl
from jax.experimental.pallas import tpu as pltpu
from jax.experimental.pallas import tpu_sc as plsc

B = 512
NUM_EDGES = 8192
NUM_NODES = 4096
PRE_LEN = 4
NUM_DIR = 8
MAX_ADJ = 16

NC = 2
NS = 16
L = 16
NW = NC * NS
RPW = B // NW

_BB = 64


def _tc_stats_body(x_ref, d_ref, mx_ref, lse_ref, am_ref, vd_ref):
    ms, lses, ams, vds = [], [], [], []
    for d in range(PRE_LEN):
        x = x_ref[:, d * NUM_EDGES:(d + 1) * NUM_EDGES]
        m = jnp.max(x, axis=1)
        lses.append(jnp.log(jnp.sum(jnp.exp(x - m[:, None]), axis=1))[:, None])
        ms.append(m[:, None])
        xd = d_ref[:, d * NUM_DIR:(d + 1) * NUM_DIR]
        md = jnp.max(xd, axis=1)
        vds.append(-jnp.log(jnp.sum(jnp.exp(xd - md[:, None]), axis=1))[:, None])
        lane = lax.broadcasted_iota(jnp.int32, xd.shape, 1)
        am = jnp.min(jnp.where(xd == md[:, None], lane, jnp.int32(NUM_DIR)),
                     axis=1)
        ams.append(am[:, None])
    mx_ref[...] = jnp.concatenate(ms, axis=1)
    lse_ref[...] = jnp.concatenate(lses, axis=1)
    am_ref[...] = jnp.concatenate(ams, axis=1)
    vd_ref[...] = jnp.concatenate(vds, axis=1)


def _tc_stats(pred, pred_d):
    out4 = jax.ShapeDtypeStruct((B, PRE_LEN), jnp.float32)
    out4i = jax.ShapeDtypeStruct((B, PRE_LEN), jnp.int32)
    return pl.pallas_call(
        _tc_stats_body,
        grid=(B // _BB,),
        in_specs=[
            pl.BlockSpec((_BB, PRE_LEN * NUM_EDGES), lambda i: (i, 0)),
            pl.BlockSpec((_BB, PRE_LEN * NUM_DIR), lambda i: (i, 0)),
        ],
        out_specs=[pl.BlockSpec((_BB, PRE_LEN), lambda i: (i, 0))] * 4,
        out_shape=[out4, out4, out4i, out4],
    )(pred, pred_d)


def _sc_chain_body(predf, gt0, obs, ge0, ge1, adj, padval,
                   outp_hbm, outv_hbm,
                   idx_v, en_v, naj_v, pidx_v, pval_v, najm_v,
                   padv_v, outp_v, outv_v, sem, sem2):
    c = lax.axis_index("c")
    s = lax.axis_index("s")
    wid = s * NC + c
    base = wid * RPW

    pltpu.sync_copy(padval, padv_v)
    pltpu.sync_copy(gt0.at[pl.ds(base, RPW)], idx_v)
    pltpu.async_copy(ge0.at[idx_v], en_v, sem).wait()
    en_v[...] = en_v[...] - 1
    pltpu.sync_copy(obs.at[pl.ds(base, RPW)], idx_v)
    last_pred = idx_v[...]
    padv = padv_v[...]
    lanes = lax.iota(jnp.int32, L)
    bvec = base + lanes

    for dim in range(PRE_LEN):
        pltpu.async_copy(adj.at[en_v], naj_v, sem).wait()
        for j in range(MAX_ADJ):
            col = plsc.load_gather(
                naj_v, [lanes, jnp.full((L,), j, jnp.int32)])
            pad = (col == jnp.int32(NUM_EDGES)) | (col == last_pred)
            najm = jnp.where(pad, jnp.int32(NUM_EDGES), col)
            a, off = j // 8, (j % 8) * L
            najm_v[a, pl.ds(off, L)] = najm
            cc = jnp.int32(dim * NUM_EDGES) + jnp.where(pad, jnp.int32(0), col)
            flat = (((bvec >> 3) << 18) | ((cc >> 7) << 10)
                    | ((bvec & 7) << 7) | (cc & 127))
            pidx_v[a, pl.ds(off, L)] = flat
        c0 = pltpu.async_copy(predf.at[pidx_v.at[0]], pval_v.at[0], sem)
        c1 = pltpu.async_copy(predf.at[pidx_v.at[1]], pval_v.at[1], sem2)
        c0.wait()
        c1.wait()
        best_v = None
        best_n = None
        for j in range(MAX_ADJ):
            a, off = j // 8, (j % 8) * L
            najm = najm_v[a, pl.ds(off, L)]
            v = pval_v[a, pl.ds(off, L)]
            v = jnp.where(najm == jnp.int32(NUM_EDGES), -jnp.inf, v)
            if j == 0:
                best_v, best_n = v, najm
            else:
                upd = v > best_v
                best_v = jnp.where(upd, v, best_v)
                best_n = jnp.where(upd, najm, best_n)
        cur = jnp.where(best_n == jnp.int32(NUM_EDGES), padv, best_n)
        oidx = lanes * jnp.int32(PRE_LEN) + jnp.int32(dim)
        plsc.store_scatter(outp_v, [oidx], cur)
        plsc.store_scatter(outv_v, [oidx], best_v)
        last_pred = cur
        if dim != PRE_LEN - 1:
            idx_v[...] = cur
            pltpu.async_copy(ge1.at[idx_v], en_v, sem).wait()
            en_v[...] = en_v[...] - 1

    pltpu.sync_copy(outp_v, outp_hbm.at[pl.ds(base * PRE_LEN, RPW * PRE_LEN)])
    pltpu.sync_copy(outv_v, outv_hbm.at[pl.ds(base * PRE_LEN, RPW * PRE_LEN)])


def _sc_chain(predf, gt0, obs, ge0, ge1, adj, padval):
    mesh = plsc.VectorSubcoreMesh(core_axis_name="c", subcore_axis_name="s",
                                  num_cores=NC, num_subcores=NS)
    fn = pl.kernel(
        _sc_chain_body,
        compiler_params=pltpu.CompilerParams(needs_layout_passes=False,
                                             use_tc_tiling_on_sc=False),
        out_type=[
            jax.ShapeDtypeStruct((B * PRE_LEN,), jnp.int32),
            jax.ShapeDtypeStruct((B * PRE_LEN,), jnp.float32),
        ],
        mesh=mesh,
        scratch_types=[
            pltpu.VMEM((RPW,), jnp.int32),
            pltpu.VMEM((RPW,), jnp.int32),
            pltpu.VMEM((RPW, MAX_ADJ), jnp.int32),
            pltpu.VMEM((2, 128), jnp.int32),
            pltpu.VMEM((2, 128), jnp.float32),
            pltpu.VMEM((2, 128), jnp.int32),
            pltpu.VMEM((L,), jnp.int32),
            pltpu.VMEM((RPW * PRE_LEN,), jnp.int32),
            pltpu.VMEM((RPW * PRE_LEN,), jnp.float32),
            pltpu.SemaphoreType.DMA,
            pltpu.SemaphoreType.DMA,
        ],
    )
    return fn(predf, gt0, obs, ge0, ge1, adj, padval)


def kernel(pred, pred_d, gt, direction_gt, obs, graph_edges, node_adj_edges,
           offset):
    mx, lse, am, vd = _tc_stats(pred, pred_d)

    width = PRE_LEN * NUM_EDGES
    predf = (pred.reshape(B // 8, 8, width // 128, 128)
             .transpose(0, 2, 1, 3).reshape(B * width))
    gt0 = gt[:, 0].astype(jnp.int32)
    ge0 = graph_edges[:, 0].astype(jnp.int32)
    ge1 = graph_edges[:, 1].astype(jnp.int32)
    adj = node_adj_edges.astype(jnp.int32)
    padval = jnp.full((L,), NUM_EDGES, jnp.int32) - offset.astype(jnp.int32)[0]
    predT, rawT = _sc_chain(predf, gt0, obs.astype(jnp.int32), ge0, ge1, adj,
                            padval)

    prediction = predT.reshape(B, PRE_LEN)
    values = (rawT.reshape(B, PRE_LEN) - mx) - lse
    prediction_d = am
    values_d = vd
    return prediction, prediction_d, values, values_d
